# baseline (speedup 1.0000x reference)
import functools

import jax
import jax.numpy as jnp
from jax import lax
from jax.experimental import pallas as pl
from jax.experimental.pallas import tpu as pltpu
from jax.experimental.pallas import tpu_sc as plsc

N_EXPERTS = 64
TOPK = 8
BLOCK_COLS = 1024
NC = 2
NS = 16
NW = NC * NS


def _score_kernel(x_ref, w_ref, b_ref, key_ref):
    s = lax.dot_general(w_ref[...], x_ref[...], (((1,), (1,)), ((), ())),
                        preferred_element_type=jnp.float32)
    s = s + b_ref[...]
    iota = lax.broadcasted_iota(jnp.int32, s.shape, 0)
    b = lax.bitcast_convert_type(s, jnp.int32)
    low = jnp.where(b >= 0, (N_EXPERTS - 1) - iota, iota)
    key_ref[...] = lax.bitcast_convert_type((b & ~(N_EXPERTS - 1)) | low,
                                            jnp.float32)


def _unpack(key):
    b = lax.bitcast_convert_type(key, jnp.int32)
    low = b & (N_EXPERTS - 1)
    idx = jnp.where(b >= 0, (N_EXPERTS - 1) - low, low)
    val = lax.bitcast_convert_type(b & ~(N_EXPERTS - 1), jnp.float32)
    return val, idx


def _topk_sc_kernel(keys_hbm, w_hbm, i_hbm, keys_v, w_v, i_v):
    rows = 8192 // NW
    groups = rows // 16
    wid = lax.axis_index("s") * NC + lax.axis_index("c")
    base = wid * rows
    pltpu.sync_copy(keys_hbm.at[:, pl.ds(base, rows)], keys_v)

    def body(g, carry):
        sl = pl.ds(g * 16, 16)
        cur = [keys_v[e, sl] for e in range(N_EXPERTS)]
        tops = []
        for k in range(TOPK):
            m = cur[0]
            for e in range(1, N_EXPERTS):
                m = jnp.maximum(m, cur[e])
            tops.append(m)
            if k < TOPK - 1:
                cur = [jnp.where(c == m, -jnp.inf, c) for c in cur]
        vals, idxs = zip(*(_unpack(t) for t in tops))
        es = [jnp.exp(v - vals[0]) for v in vals]
        tot = es[0]
        for k in range(1, TOPK):
            tot = tot + es[k]
        for k in range(TOPK):
            w_v[k, sl] = es[k] / tot
            i_v[k, sl] = idxs[k]
        return carry

    lax.fori_loop(0, groups, body, 0)
    pltpu.sync_copy(w_v, w_hbm.at[:, pl.ds(base, rows)])
    pltpu.sync_copy(i_v, i_hbm.at[:, pl.ds(base, rows)])


def kernel(x, weight, bias, target_dist):
    del target_dist
    n_tokens, dim = x.shape
    b2 = bias.reshape(N_EXPERTS, 1)
    grid = (n_tokens // BLOCK_COLS,)
    keys_t = pl.pallas_call(
        _score_kernel,
        grid=grid,
        in_specs=[
            pl.BlockSpec((BLOCK_COLS, dim), lambda i: (i, 0)),
            pl.BlockSpec((N_EXPERTS, dim), lambda i: (0, 0)),
            pl.BlockSpec((N_EXPERTS, 1), lambda i: (0, 0)),
        ],
        out_specs=pl.BlockSpec((N_EXPERTS, BLOCK_COLS), lambda i: (0, i)),
        out_shape=jax.ShapeDtypeStruct((N_EXPERTS, n_tokens), jnp.float32),
    )(x, weight, b2)

    rows = n_tokens // NW
    sc_topk = functools.partial(
        pl.kernel,
        mesh=plsc.VectorSubcoreMesh(core_axis_name="c", subcore_axis_name="s"),
        out_type=[
            jax.ShapeDtypeStruct((TOPK, n_tokens), jnp.float32),
            jax.ShapeDtypeStruct((TOPK, n_tokens), jnp.int32),
        ],
        scratch_types=[
            pltpu.VMEM((N_EXPERTS, rows), jnp.float32),
            pltpu.VMEM((TOPK, rows), jnp.float32),
            pltpu.VMEM((TOPK, rows), jnp.int32),
        ],
    )(_topk_sc_kernel)
    w_t, i_t = sc_topk(keys_t)
    return (w_t.T, i_t.T)
